# baseline (speedup 1.0000x reference)
import jax
import jax.numpy as jnp
from jax.experimental import pallas as pl
from jax.experimental.pallas import tpu as pltpu

N_F = 20000
N_C = 256
D_F = 256
K_PATCH = 64
NUM_CORR = 256
SINK_ITERS = 100
NEG = -1e4

BBLK = 32
NPAD = 72
MPAD = 128


def _ot_kernel(alpha_ref, rfeat_ref, sfeat_ref, rowm_ref, colm_ref, out_ref):
    a = alpha_ref[0]
    rf = rfeat_ref[...]
    sf = sfeat_ref[...]
    scores = jax.lax.dot_general(
        rf, sf, (((2,), (2,)), ((0,), (0,))),
        preferred_element_type=jnp.float32) * (1.0 / 16.0)

    rowm = rowm_ref[...][:, :, None]
    colm = colm_ref[...][:, None, :]
    ii = jax.lax.broadcasted_iota(jnp.int32, (1, NPAD, 1), 1)
    jj = jax.lax.broadcasted_iota(jnp.int32, (1, 1, MPAD), 2)
    edge = (ii == K_PATCH) | (jj == K_PATCH)
    Z = jnp.where(edge, a, scores)
    Z = jnp.where((rowm > 0.5) & (colm > 0.5), Z, NEG)

    nr = jnp.sum(rowm, axis=1, keepdims=True) - 1.0
    nc = jnp.sum(colm, axis=2, keepdims=True) - 1.0
    norm = -jnp.log(nr + nc + 1e-12)
    log_mu = jnp.where(ii == K_PATCH, jnp.log(nc + 1e-12) + norm,
                       jnp.where(rowm > 0.5, norm, NEG))
    log_nu = jnp.where(jj == K_PATCH, jnp.log(nr + 1e-12) + norm,
                       jnp.where(colm > 0.5, norm, NEG))

    def lse(x, axis):
        m = jnp.max(x, axis=axis, keepdims=True)
        return jnp.log(jnp.sum(jnp.exp(x - m), axis=axis, keepdims=True)) + m

    def body(_, carry):
        u, v = carry
        u = log_mu - lse(Z + v, 2)
        v = log_nu - lse(Z + u, 1)
        return u, v

    u0 = jnp.zeros_like(log_mu)
    v0 = jnp.zeros_like(log_nu)
    u, v = jax.lax.fori_loop(0, SINK_ITERS, body, (u0, v0))
    out = Z + u + v
    out_ref[...] = out[:, :K_PATCH + 1, :K_PATCH + 1]


CBLK = 64
FINF = 3.4e38
IBIG = 2**30


def _knn_kernel(pc_ref, pft_ref, knn_ref, mn_ref, ci_ref):
    b = pl.program_id(0)
    pc = pc_ref[...]
    d2 = None
    for d in range(3):
        diff = pc[:, d:d + 1] - pft_ref[d:d + 1, :]
        sq = diff * diff
        d2 = sq if d2 is None else d2 + sq

    c_iota = jax.lax.broadcasted_iota(jnp.int32, (CBLK, N_F), 0) + b * CBLK
    mn = jnp.min(d2, axis=0, keepdims=True)
    ci = jnp.min(jnp.where(d2 == mn, c_iota, jnp.int32(IBIG)), axis=0, keepdims=True)
    mn_ref[...] = mn.reshape(1, 1, N_F)
    ci_ref[...] = ci.reshape(1, 1, N_F)

    vals = -d2
    li = jax.lax.broadcasted_iota(jnp.int32, (CBLK, N_F), 1)
    col = jax.lax.broadcasted_iota(jnp.int32, (CBLK, K_PATCH), 1)

    def body(t, carry):
        m_prev, i_prev, acc = carry
        elig = (vals < m_prev) | ((vals == m_prev) & (li > i_prev))
        cand = jnp.where(elig, vals, jnp.float32(-FINF))
        m = jnp.max(cand, axis=1, keepdims=True)
        idx = jnp.min(jnp.where(cand == m, li, jnp.int32(IBIG)), axis=1, keepdims=True)
        acc = jnp.where(col == t, idx, acc)
        return m, idx, acc

    m0 = jnp.full((CBLK, 1), FINF, jnp.float32)
    i0 = jnp.full((CBLK, 1), -1, jnp.int32)
    acc0 = jnp.zeros((CBLK, K_PATCH), jnp.int32)
    _, _, acc = jax.lax.fori_loop(0, K_PATCH, body, (m0, i0, acc0))
    knn_ref[...] = acc


def _partition(points_f, points_c, k):
    nc = points_c.shape[0]
    pft = points_f.T
    knn_indices, mn4, ci4 = pl.pallas_call(
        _knn_kernel,
        grid=(nc // CBLK,),
        in_specs=[
            pl.BlockSpec((CBLK, 3), lambda b: (b, 0)),
            pl.BlockSpec((3, N_F), lambda b: (0, 0)),
        ],
        out_specs=[
            pl.BlockSpec((CBLK, K_PATCH), lambda b: (b, 0)),
            pl.BlockSpec((1, 1, N_F), lambda b: (b, 0, 0)),
            pl.BlockSpec((1, 1, N_F), lambda b: (b, 0, 0)),
        ],
        out_shape=[
            jax.ShapeDtypeStruct((nc, K_PATCH), jnp.int32),
            jax.ShapeDtypeStruct((nc // CBLK, 1, N_F), jnp.float32),
            jax.ShapeDtypeStruct((nc // CBLK, 1, N_F), jnp.int32),
        ],
    )(points_c, pft)
    mn4 = mn4[:, 0, :]
    ci4 = ci4[:, 0, :]
    blk = jnp.argmin(mn4, axis=0)
    point_to_node = jnp.take_along_axis(ci4, blk[None, :], axis=0)[0]
    node_masks = jnp.zeros(nc, dtype=bool).at[point_to_node].set(True)
    knn_masks = point_to_node[knn_indices] == jnp.arange(nc)[:, None]
    return point_to_node, node_masks, knn_indices, knn_masks


def kernel(ref_points_f, src_points_f, ref_feats_f, src_feats_f,
           ref_points_c, src_points_c, ref_feats_c, src_feats_c, alpha):
    _, ref_node_masks, ref_knn_idx, ref_knn_masks = _partition(
        ref_points_f, ref_points_c, K_PATCH)
    _, src_node_masks, src_knn_idx, src_knn_masks = _partition(
        src_points_f, src_points_c, K_PATCH)

    ref_n = ref_feats_c / (jnp.linalg.norm(ref_feats_c, axis=1, keepdims=True) + 1e-12)
    src_n = src_feats_c / (jnp.linalg.norm(src_feats_c, axis=1, keepdims=True) + 1e-12)
    dist = 2.0 - 2.0 * (ref_n @ src_n.T)
    s = jnp.exp(-dist)
    s = jnp.where(ref_node_masks[:, None] & src_node_masks[None, :], s, 0.0)
    ref_ms = s / (s.sum(1, keepdims=True) + 1e-12)
    src_ms = s / (s.sum(0, keepdims=True) + 1e-12)
    s = ref_ms * src_ms
    _, corr_idx = jax.lax.top_k(s.reshape(-1), NUM_CORR)
    ref_ci = corr_idx // N_C
    src_ci = corr_idx % N_C

    rknn = ref_knn_idx[ref_ci]
    sknn = src_knn_idx[src_ci]
    rmask = ref_knn_masks[ref_ci]
    smask = src_knn_masks[src_ci]
    ref_pad = jnp.concatenate([ref_feats_f, jnp.zeros_like(ref_feats_f[:1])], 0)
    src_pad = jnp.concatenate([src_feats_f, jnp.zeros_like(src_feats_f[:1])], 0)
    rfeats = ref_pad[rknn]
    sfeats = src_pad[sknn]

    rf_pad = jnp.pad(rfeats, ((0, 0), (0, NPAD - K_PATCH), (0, 0)))
    sf_pad = jnp.pad(sfeats, ((0, 0), (0, MPAD - K_PATCH), (0, 0)))
    rowm = jnp.pad(rmask.astype(jnp.float32), ((0, 0), (0, NPAD - K_PATCH)))
    rowm = rowm.at[:, K_PATCH].set(1.0)
    colm = jnp.pad(smask.astype(jnp.float32), ((0, 0), (0, MPAD - K_PATCH)))
    colm = colm.at[:, K_PATCH].set(1.0)

    out = pl.pallas_call(
        _ot_kernel,
        grid=(NUM_CORR // BBLK,),
        in_specs=[
            pl.BlockSpec(memory_space=pltpu.SMEM),
            pl.BlockSpec((BBLK, NPAD, D_F), lambda b: (b, 0, 0)),
            pl.BlockSpec((BBLK, MPAD, D_F), lambda b: (b, 0, 0)),
            pl.BlockSpec((BBLK, NPAD), lambda b: (b, 0)),
            pl.BlockSpec((BBLK, MPAD), lambda b: (b, 0)),
        ],
        out_specs=pl.BlockSpec((BBLK, K_PATCH + 1, K_PATCH + 1),
                               lambda b: (b, 0, 0)),
        out_shape=jax.ShapeDtypeStruct((NUM_CORR, K_PATCH + 1, K_PATCH + 1),
                                       jnp.float32),
    )(alpha, rf_pad, sf_pad, rowm, colm)
    return out


# speedup vs baseline: 1.0166x; 1.0166x over previous
import jax
import jax.numpy as jnp
from jax.experimental import pallas as pl
from jax.experimental.pallas import tpu as pltpu

N_F = 20000
N_C = 256
D_F = 256
K_PATCH = 64
NUM_CORR = 256
SINK_ITERS = 100
NEG = -1e4

BBLK = 32
NPAD = 72
MPAD = 128


def _ot_kernel(alpha_ref, rfeat_ref, sfeat_ref, rowm_ref, colm_ref, out_ref):
    a = alpha_ref[0]
    rf = rfeat_ref[...]
    sf = sfeat_ref[...]
    scores = jax.lax.dot_general(
        rf, sf, (((2,), (2,)), ((0,), (0,))),
        preferred_element_type=jnp.float32) * (1.0 / 16.0)

    rowm = rowm_ref[...][:, :, None]
    colm = colm_ref[...][:, None, :]
    ii = jax.lax.broadcasted_iota(jnp.int32, (1, NPAD, 1), 1)
    jj = jax.lax.broadcasted_iota(jnp.int32, (1, 1, MPAD), 2)
    edge = (ii == K_PATCH) | (jj == K_PATCH)
    rv = rowm > 0.5
    cv = colm > 0.5
    Z = jnp.where(rv & cv, jnp.where(edge, a, scores),
                  jnp.where(rv == cv, -NEG, 0.0))

    nr = jnp.sum(rowm, axis=1, keepdims=True) - 1.0
    nc = jnp.sum(colm, axis=2, keepdims=True) - 1.0
    norm = -jnp.log(nr + nc + 1e-12)
    log_mu = jnp.where(ii == K_PATCH, jnp.log(nc + 1e-12) + norm,
                       jnp.where(rowm > 0.5, norm, NEG))
    log_nu = jnp.where(jj == K_PATCH, jnp.log(nr + 1e-12) + norm,
                       jnp.where(colm > 0.5, norm, NEG))

    def lse(x, axis):
        s = jnp.sum(jnp.exp(x), axis=axis, keepdims=True)
        return jnp.log(jnp.maximum(s, 1e-30))

    def body(_, carry):
        u, v = carry
        u = log_mu - lse(Z + v, 2)
        v = log_nu - lse(Z + u, 1)
        return u, v

    u0 = jnp.zeros_like(log_mu)
    v0 = jnp.where(cv, 0.0, NEG) + jnp.zeros_like(log_nu)
    u, v = jax.lax.fori_loop(0, SINK_ITERS, body, (u0, v0))
    out = Z + u + v
    out_ref[...] = out[:, :K_PATCH + 1, :K_PATCH + 1]


CBLK = 64
FINF = 3.4e38
IBIG = 2**30


def _knn_kernel(pc_ref, pft_ref, knn_ref, mn_ref, ci_ref):
    b = pl.program_id(0)
    pc = pc_ref[...]
    d2 = None
    for d in range(3):
        diff = pc[:, d:d + 1] - pft_ref[d:d + 1, :]
        sq = diff * diff
        d2 = sq if d2 is None else d2 + sq

    c_iota = jax.lax.broadcasted_iota(jnp.int32, (CBLK, N_F), 0) + b * CBLK
    mn = jnp.min(d2, axis=0, keepdims=True)
    ci = jnp.min(jnp.where(d2 == mn, c_iota, jnp.int32(IBIG)), axis=0, keepdims=True)
    mn_ref[...] = mn.reshape(1, 1, N_F)
    ci_ref[...] = ci.reshape(1, 1, N_F)

    vals = -d2
    li = jax.lax.broadcasted_iota(jnp.int32, (CBLK, N_F), 1)
    col = jax.lax.broadcasted_iota(jnp.int32, (CBLK, K_PATCH), 1)

    def body(t, carry):
        cand, acc = carry
        m = jnp.max(cand, axis=1, keepdims=True)
        idx = jnp.min(jnp.where(cand == m, li, jnp.int32(IBIG)), axis=1, keepdims=True)
        acc = jnp.where(col == t, idx, acc)
        cand = jnp.where(li == idx, jnp.float32(-FINF), cand)
        return cand, acc

    acc0 = jnp.zeros((CBLK, K_PATCH), jnp.int32)
    _, acc = jax.lax.fori_loop(0, K_PATCH, body, (vals, acc0))
    knn_ref[...] = acc


def _partition(points_f, points_c, k):
    nc = points_c.shape[0]
    pft = points_f.T
    knn_indices, mn4, ci4 = pl.pallas_call(
        _knn_kernel,
        grid=(nc // CBLK,),
        in_specs=[
            pl.BlockSpec((CBLK, 3), lambda b: (b, 0)),
            pl.BlockSpec((3, N_F), lambda b: (0, 0)),
        ],
        out_specs=[
            pl.BlockSpec((CBLK, K_PATCH), lambda b: (b, 0)),
            pl.BlockSpec((1, 1, N_F), lambda b: (b, 0, 0)),
            pl.BlockSpec((1, 1, N_F), lambda b: (b, 0, 0)),
        ],
        out_shape=[
            jax.ShapeDtypeStruct((nc, K_PATCH), jnp.int32),
            jax.ShapeDtypeStruct((nc // CBLK, 1, N_F), jnp.float32),
            jax.ShapeDtypeStruct((nc // CBLK, 1, N_F), jnp.int32),
        ],
    )(points_c, pft)
    mn4 = mn4[:, 0, :]
    ci4 = ci4[:, 0, :]
    blk = jnp.argmin(mn4, axis=0)
    point_to_node = jnp.take_along_axis(ci4, blk[None, :], axis=0)[0]
    node_masks = jnp.zeros(nc, dtype=bool).at[point_to_node].set(True)
    knn_masks = point_to_node[knn_indices] == jnp.arange(nc)[:, None]
    return point_to_node, node_masks, knn_indices, knn_masks


def kernel(ref_points_f, src_points_f, ref_feats_f, src_feats_f,
           ref_points_c, src_points_c, ref_feats_c, src_feats_c, alpha):
    _, ref_node_masks, ref_knn_idx, ref_knn_masks = _partition(
        ref_points_f, ref_points_c, K_PATCH)
    _, src_node_masks, src_knn_idx, src_knn_masks = _partition(
        src_points_f, src_points_c, K_PATCH)

    ref_n = ref_feats_c / (jnp.linalg.norm(ref_feats_c, axis=1, keepdims=True) + 1e-12)
    src_n = src_feats_c / (jnp.linalg.norm(src_feats_c, axis=1, keepdims=True) + 1e-12)
    dist = 2.0 - 2.0 * (ref_n @ src_n.T)
    s = jnp.exp(-dist)
    s = jnp.where(ref_node_masks[:, None] & src_node_masks[None, :], s, 0.0)
    ref_ms = s / (s.sum(1, keepdims=True) + 1e-12)
    src_ms = s / (s.sum(0, keepdims=True) + 1e-12)
    s = ref_ms * src_ms
    _, corr_idx = jax.lax.top_k(s.reshape(-1), NUM_CORR)
    ref_ci = corr_idx // N_C
    src_ci = corr_idx % N_C

    rknn = ref_knn_idx[ref_ci]
    sknn = src_knn_idx[src_ci]
    rmask = ref_knn_masks[ref_ci]
    smask = src_knn_masks[src_ci]
    ref_pad = jnp.concatenate([ref_feats_f, jnp.zeros_like(ref_feats_f[:1])], 0)
    src_pad = jnp.concatenate([src_feats_f, jnp.zeros_like(src_feats_f[:1])], 0)
    rfeats = ref_pad[rknn]
    sfeats = src_pad[sknn]

    rf_pad = jnp.pad(rfeats, ((0, 0), (0, NPAD - K_PATCH), (0, 0)))
    sf_pad = jnp.pad(sfeats, ((0, 0), (0, MPAD - K_PATCH), (0, 0)))
    rowm = jnp.pad(rmask.astype(jnp.float32), ((0, 0), (0, NPAD - K_PATCH)))
    rowm = rowm.at[:, K_PATCH].set(1.0)
    colm = jnp.pad(smask.astype(jnp.float32), ((0, 0), (0, MPAD - K_PATCH)))
    colm = colm.at[:, K_PATCH].set(1.0)

    out = pl.pallas_call(
        _ot_kernel,
        grid=(NUM_CORR // BBLK,),
        in_specs=[
            pl.BlockSpec(memory_space=pltpu.SMEM),
            pl.BlockSpec((BBLK, NPAD, D_F), lambda b: (b, 0, 0)),
            pl.BlockSpec((BBLK, MPAD, D_F), lambda b: (b, 0, 0)),
            pl.BlockSpec((BBLK, NPAD), lambda b: (b, 0)),
            pl.BlockSpec((BBLK, MPAD), lambda b: (b, 0)),
        ],
        out_specs=pl.BlockSpec((BBLK, K_PATCH + 1, K_PATCH + 1),
                               lambda b: (b, 0, 0)),
        out_shape=jax.ShapeDtypeStruct((NUM_CORR, K_PATCH + 1, K_PATCH + 1),
                                       jnp.float32),
    )(alpha, rf_pad, sf_pad, rowm, colm)
    return out
